# baseline (speedup 1.0000x reference)
import functools

import jax
import jax.numpy as jnp
from jax import lax
from jax.experimental import pallas as pl
from jax.experimental.pallas import tpu as pltpu
from jax.experimental.pallas import tpu_sc as plsc

B = 16384
N_OBJ = 64
N_ACT = 8
P_SPACE = 0.1
EXIST_THR = 0.8

NC, NS, L = 2, 16, 16
NW = NC * NS
ROWS_PER_W = B // NW
NB = 16
TILES = ROWS_PER_W // NB
NM = N_OBJ // L


def _sc_body(x_hbm, act_hbm, mask_hbm, ap_hbm, pv_hbm,
             gat_a, gat_b, mask_v, mexp_v, act_v, sat_v,
             ap_v, pv_v, sem_a, sem_b):
    wid = lax.axis_index("s") * NC + lax.axis_index("c")
    base_row = wid * ROWS_PER_W

    pltpu.sync_copy(mask_hbm, mask_v)
    pltpu.sync_copy(act_hbm, act_v)

    iota = lax.iota(jnp.int32, L)

    for m in range(NM):
        chunk = mask_v[pl.ds(m * L, L)]
        for j in range(L):
            mexp_v[pl.ds((m * L + j) * L, L)] = jnp.broadcast_to(
                chunk[j], (L,))

    act = act_v[...]
    an = act / (act + 1e-20)
    half = (iota >= 8).astype(jnp.int32)

    def fire(t, gat_v, sem):
        b0 = base_row + t * NB
        for m in range(NM):
            pltpu.async_copy(
                x_hbm.at[pl.ds(b0, NB), pl.ds(L * m, L), m],
                gat_v.at[m], sem)

    def drain(t, gat_v, sem):
        b0 = base_row + t * NB
        for m in range(NM):
            pltpu.make_async_copy(
                x_hbm.at[pl.ds(b0, NB), pl.ds(L * m, L), m],
                gat_v.at[m], sem).wait()

    zero = jnp.zeros((L,), jnp.int32)
    one = jnp.full((L,), 1, jnp.int32)

    def compute(t, gat_v):
        a_val = plsc.load_gather(gat_v, [zero, iota, zero, zero])
        b_val = plsc.load_gather(gat_v, [zero, iota, one, zero])
        p = jnp.abs(a_val - b_val)
        acc = p < P_SPACE
        for i in range(N_OBJ):
            m, q = i // L, i % L
            qv = jnp.full((L,), q, jnp.int32)
            diag = plsc.load_gather(gat_v, [jnp.full((L,), m, jnp.int32),
                                            iota, qv, qv])
            m_i = mexp_v[pl.ds(i * L, L)] > 0
            acc = acc & ((diag > EXIST_THR) == m_i)
        satf = jnp.where(acc, 1.0, 0.0).astype(jnp.float32)

        pv_v[pl.ds(t * NB, NB)] = p
        sat_v[...] = satf
        for pair in range(NB // 2):
            sel = half + 2 * pair
            ap_v[pl.ds(t * NB * N_ACT + pair * L, L)] = (
                plsc.load_gather(sat_v, [sel]) * an)

    fire(0, gat_a, sem_a)

    def step(s, carry):
        ta = 2 * s
        tb = 2 * s + 1
        fire(tb, gat_b, sem_b)
        drain(ta, gat_a, sem_a)
        compute(ta, gat_a)
        ta_next = jnp.minimum(ta + 2, TILES - 1)
        fire(ta_next, gat_a, sem_a)
        drain(tb, gat_b, sem_b)
        compute(tb, gat_b)
        return carry

    lax.fori_loop(0, TILES // 2, step, 0, unroll=False)
    drain(TILES - 1, gat_a, sem_a)

    pltpu.sync_copy(pv_v, pv_hbm.at[pl.ds(base_row, ROWS_PER_W)])
    pltpu.sync_copy(ap_v, ap_hbm.at[pl.ds(base_row * N_ACT,
                                          ROWS_PER_W * N_ACT)])


@jax.jit
def _run(x6, act2, mask_i32):
    mesh = plsc.VectorSubcoreMesh(core_axis_name="c", subcore_axis_name="s")
    f = functools.partial(
        pl.kernel,
        mesh=mesh,
        compiler_params=pltpu.CompilerParams(needs_layout_passes=False,
                                             use_tc_tiling_on_sc=False),
        out_type=[
            jax.ShapeDtypeStruct((B * N_ACT,), jnp.float32),
            jax.ShapeDtypeStruct((B,), jnp.float32),
        ],
        scratch_types=[
            pltpu.VMEM((NM, NB, L, L), jnp.float32),
            pltpu.VMEM((NM, NB, L, L), jnp.float32),
            pltpu.VMEM((N_OBJ,), jnp.int32),
            pltpu.VMEM((N_OBJ * L,), jnp.int32),
            pltpu.VMEM((L,), jnp.float32),
            pltpu.VMEM((L,), jnp.float32),
            pltpu.VMEM((ROWS_PER_W * N_ACT,), jnp.float32),
            pltpu.VMEM((ROWS_PER_W,), jnp.float32),
            pltpu.SemaphoreType.DMA,
            pltpu.SemaphoreType.DMA,
        ],
    )(_sc_body)
    return f(x6, act2, mask_i32)


def kernel(x, action, mask):
    x6 = x.reshape(B, N_OBJ, NM, L)
    act2 = jnp.concatenate([action, action]).astype(jnp.float32)
    mask_i32 = mask.astype(jnp.int32)
    ap_flat, pv = _run(x6, act2, mask_i32)
    return (ap_flat.reshape(B, N_ACT), pv)


# speedup vs baseline: 2.6408x; 2.6408x over previous
import functools

import jax
import jax.numpy as jnp
from jax import lax
from jax.experimental import pallas as pl
from jax.experimental.pallas import tpu as pltpu
from jax.experimental.pallas import tpu_sc as plsc

B = 16384
N_OBJ = 64
N_ACT = 8
P_SPACE = 0.1
EXIST_THR = 0.8

NC, NS, L = 2, 16, 16
NW = NC * NS
ROWS_PER_W = B // NW
NB = 16
TILES = ROWS_PER_W // NB
GROW = 256
GAT = NB * GROW


def _sc_body(x_hbm, act_hbm, mask_hbm, ap_hbm, pv_hbm,
             gat_v, mask_v, mexp_v, act_v, sat_v, ap_v, pv_v, sem):
    wid = lax.axis_index("s") * NC + lax.axis_index("c")
    base_row = wid * ROWS_PER_W

    pltpu.sync_copy(mask_hbm, mask_v)
    pltpu.sync_copy(act_hbm, act_v)

    iota = lax.iota(jnp.int32, L)
    r256 = iota * GROW

    for m in range(N_OBJ // L):
        chunk = mask_v[pl.ds(m * L, L)]
        for j in range(L):
            mexp_v[pl.ds((m * L + j) * L, L)] = jnp.broadcast_to(
                chunk[j], (L,))

    act = act_v[...]
    an = act / (act + 1e-20)
    half = (iota >= 8).astype(jnp.int32)
    zero = jnp.zeros((L,), jnp.int32)

    def tile(t, carry):
        b0 = base_row + t * NB
        pltpu.sync_copy(x_hbm.at[pl.ds(b0 * GROW, GAT)], gat_v)

        a_val = plsc.load_gather(gat_v, [r256, zero])
        b_val = plsc.load_gather(gat_v, [r256 + 4, zero])
        p = jnp.abs(a_val - b_val)
        acc = p < P_SPACE
        for i in range(N_OBJ):
            rows = r256 + (65 * i) // 16
            col = jnp.full((L,), i % 16, jnp.int32)
            diag = plsc.load_gather(gat_v, [rows, col])
            m_i = mexp_v[pl.ds(i * L, L)] > 0
            acc = acc & ((diag > EXIST_THR) == m_i)
        satf = jnp.where(acc, 1.0, 0.0).astype(jnp.float32)

        pv_v[pl.ds(t * NB, NB)] = p
        sat_v[...] = satf
        for pair in range(NB // 2):
            sel = half + 2 * pair
            ap_v[pl.ds(t * NB * N_ACT + pair * L, L)] = (
                plsc.load_gather(sat_v, [sel]) * an)
        return carry

    lax.fori_loop(0, TILES, tile, 0, unroll=False)

    pltpu.sync_copy(pv_v, pv_hbm.at[pl.ds(base_row, ROWS_PER_W)])
    pltpu.sync_copy(ap_v, ap_hbm.at[pl.ds(base_row * N_ACT,
                                          ROWS_PER_W * N_ACT)])


@jax.jit
def _run(x2, act2, mask_i32):
    mesh = plsc.VectorSubcoreMesh(core_axis_name="c", subcore_axis_name="s")
    f = functools.partial(
        pl.kernel,
        mesh=mesh,
        compiler_params=pltpu.CompilerParams(needs_layout_passes=False,
                                             use_tc_tiling_on_sc=False),
        out_type=[
            jax.ShapeDtypeStruct((B * N_ACT,), jnp.float32),
            jax.ShapeDtypeStruct((B,), jnp.float32),
        ],
        scratch_types=[
            pltpu.VMEM((GAT, L), jnp.float32),
            pltpu.VMEM((N_OBJ,), jnp.int32),
            pltpu.VMEM((N_OBJ * L,), jnp.int32),
            pltpu.VMEM((L,), jnp.float32),
            pltpu.VMEM((L,), jnp.float32),
            pltpu.VMEM((ROWS_PER_W * N_ACT,), jnp.float32),
            pltpu.VMEM((ROWS_PER_W,), jnp.float32),
            pltpu.SemaphoreType.DMA,
        ],
    )(_sc_body)
    return f(x2, act2, mask_i32)


def kernel(x, action, mask):
    x2 = x.reshape(B * 256, 16)
    act2 = jnp.concatenate([action, action]).astype(jnp.float32)
    mask_i32 = mask.astype(jnp.int32)
    ap_flat, pv = _run(x2, act2, mask_i32)
    return (ap_flat.reshape(B, N_ACT), pv)
